# Initial kernel scaffold; baseline (speedup 1.0000x reference)
import jax
import jax.numpy as jnp
from jax.experimental import pallas as pl


def kernel(sparse_inputs, dense_inputs, tables, W_sparse, b_sparse, W_dense, b_dense):
    raise NotImplementedError("write your pallas kernel here")


# speedup vs baseline: 2.2024x; 2.2024x over previous
import functools

import jax
import jax.numpy as jnp
from jax import lax
from jax.experimental import pallas as pl
from jax.experimental.pallas import tpu as pltpu
from jax.experimental.pallas import tpu_sc as plsc

N_FIELDS = 26
VOCAB = 100000
EMB_DIM = 32
BATCH = 4096

NC = 2
NS = 16
NW = NC * NS
PAIRS = BATCH * N_FIELDS // NW
ROWS_PER_STREAM = 128
N_STREAMS = PAIRS // ROWS_PER_STREAM
LANES = 16


def _sc_gather_body(idx_hbm, table_hbm, out_hbm, idx_v, rows_v, sem):
    wid = lax.axis_index("s") * NC + lax.axis_index("c")
    base = wid * PAIRS
    pltpu.sync_copy(idx_hbm.at[pl.ds(base, PAIRS)], idx_v)
    lane = lax.iota(jnp.int32, LANES)
    for t in range(PAIRS // LANES):
        p = t * LANES + lane
        off = (p % N_FIELDS) * VOCAB
        idx_v[pl.ds(t * LANES, LANES)] = idx_v[pl.ds(t * LANES, LANES)] + off
    copies = []
    for j in range(N_STREAMS):
        c = pltpu.async_copy(
            table_hbm.at[idx_v.at[pl.ds(j * ROWS_PER_STREAM, ROWS_PER_STREAM)]],
            rows_v.at[pl.ds(j * ROWS_PER_STREAM, ROWS_PER_STREAM)],
            sem,
        )
        copies.append(c)
    for c in copies:
        c.wait()
    pltpu.sync_copy(rows_v, out_hbm.at[pl.ds(base, PAIRS)])


@functools.partial(
    pl.kernel,
    mesh=plsc.VectorSubcoreMesh(core_axis_name="c", subcore_axis_name="s"),
    compiler_params=pltpu.CompilerParams(use_tc_tiling_on_sc=False),
    out_type=jax.ShapeDtypeStruct((BATCH * N_FIELDS, EMB_DIM), jnp.float32),
    scratch_types=[
        pltpu.VMEM((PAIRS,), jnp.int32),
        pltpu.VMEM((PAIRS, EMB_DIM), jnp.float32),
        pltpu.SemaphoreType.DMA,
    ],
)
def _sc_gather(idx_hbm, table_hbm, out_hbm, idx_v, rows_v, sem):
    _sc_gather_body(idx_hbm, table_hbm, out_hbm, idx_v, rows_v, sem)


def _mm_body(x_ref, ws_ref, bs_ref, d_ref, wd_ref, bd_ref, so_ref, do_ref):
    so_ref[...] = lax.dot_general(
        x_ref[...], ws_ref[...], (((1,), (1,)), ((), ())),
        preferred_element_type=jnp.float32) + bs_ref[...]
    do_ref[...] = lax.dot_general(
        d_ref[...], wd_ref[...], (((1,), (1,)), ((), ())),
        preferred_element_type=jnp.float32) + bd_ref[...]


def _tc_matmuls(concat, w_s, b_s, dense, w_d, b_d):
    blk = 512
    grid = (BATCH // blk,)
    f_in = concat.shape[1]
    d_in = dense.shape[1]
    return pl.pallas_call(
        _mm_body,
        grid=grid,
        in_specs=[
            pl.BlockSpec((blk, f_in), lambda i: (i, 0)),
            pl.BlockSpec((w_s.shape[0], f_in), lambda i: (0, 0)),
            pl.BlockSpec((1, w_s.shape[0]), lambda i: (0, 0)),
            pl.BlockSpec((blk, d_in), lambda i: (i, 0)),
            pl.BlockSpec((w_d.shape[0], d_in), lambda i: (0, 0)),
            pl.BlockSpec((1, w_d.shape[0]), lambda i: (0, 0)),
        ],
        out_specs=[
            pl.BlockSpec((blk, w_s.shape[0]), lambda i: (i, 0)),
            pl.BlockSpec((blk, w_d.shape[0]), lambda i: (i, 0)),
        ],
        out_shape=[
            jax.ShapeDtypeStruct((BATCH, w_s.shape[0]), jnp.float32),
            jax.ShapeDtypeStruct((BATCH, w_d.shape[0]), jnp.float32),
        ],
    )(concat, w_s, b_s, dense, w_d, b_d)


def kernel(sparse_inputs, dense_inputs, tables, W_sparse, b_sparse, W_dense, b_dense):
    idx_flat = sparse_inputs.astype(jnp.int32).reshape(-1)
    table_flat = tables.reshape(N_FIELDS * VOCAB, EMB_DIM)
    gathered = _sc_gather(idx_flat, table_flat)
    concat = gathered.reshape(BATCH, N_FIELDS * EMB_DIM)
    sparse_out, dense_out = _tc_matmuls(
        concat, W_sparse, b_sparse.reshape(1, -1),
        dense_inputs, W_dense, b_dense.reshape(1, -1))
    return (dense_out, sparse_out)


# speedup vs baseline: 3.8763x; 1.7601x over previous
import functools

import jax
import jax.numpy as jnp
from jax import lax
from jax.experimental import pallas as pl
from jax.experimental.pallas import tpu as pltpu
from jax.experimental.pallas import tpu_sc as plsc

N_FIELDS = 26
VOCAB = 100000
EMB_DIM = 32
BATCH = 4096

NC = 2
NS = 16
NW = NC * NS
ROWS = N_FIELDS * EMB_DIM
BW = BATCH // NW
LANES = 16
CHUNKS_PER_ROW = VOCAB // LANES
N_CHUNKS = ROWS * CHUNKS_PER_ROW


def _sc_gather_body(idx_hbm, tab_hbm, out_hbm, idx_v, idxrow_v, cidx_v,
                    staged_v, ebuf_v, sem):
    wid = lax.axis_index("s") * NC + lax.axis_index("c")
    col0 = wid * BW
    pltpu.sync_copy(idx_hbm.at[:, pl.ds(col0, BW)], idx_v)
    lane = lax.iota(jnp.int32, LANES)

    def per_field(f, _):
        vs = [idx_v[f, pl.ds(g * LANES, LANES)] for g in range(BW // LANES)]
        vc16 = [v >> 4 for v in vs]
        vr = [v & 15 for v in vs]
        base0 = (f * EMB_DIM) * CHUNKS_PER_ROW
        for d in range(EMB_DIM):
            base = base0 + d * CHUNKS_PER_ROW
            for g in range(BW // LANES):
                cidx_v[d, pl.ds(g * LANES, LANES)] = vc16[g] + base
        copies = []
        for d in range(EMB_DIM):
            copies.append(pltpu.async_copy(
                tab_hbm.at[cidx_v.at[d]],
                staged_v.at[pl.ds(d * BW, BW)],
                sem,
            ))
        pltpu.make_async_copy(
            tab_hbm.at[pl.ds(0, EMB_DIM * BW)], staged_v, sem
        ).wait()
        for d in range(EMB_DIM):
            for g in range(BW // LANES):
                chunkv = d * BW + g * LANES + lane
                ebuf_v[d, pl.ds(g * LANES, LANES)] = plsc.load_gather(
                    staged_v, [chunkv, vr[g]])
        pltpu.sync_copy(
            ebuf_v, out_hbm.at[pl.ds(f * EMB_DIM, EMB_DIM), pl.ds(col0, BW)])
        return 0

    lax.fori_loop(0, N_FIELDS, per_field, 0)


@functools.partial(
    pl.kernel,
    mesh=plsc.VectorSubcoreMesh(core_axis_name="c", subcore_axis_name="s"),
    compiler_params=pltpu.CompilerParams(
        use_tc_tiling_on_sc=False, needs_layout_passes=False),
    out_type=jax.ShapeDtypeStruct((ROWS, BATCH), jnp.float32),
    scratch_types=[
        pltpu.VMEM((N_FIELDS, BW), jnp.int32),
        pltpu.VMEM((BW,), jnp.int32),
        pltpu.VMEM((EMB_DIM, BW), jnp.int32),
        pltpu.VMEM((EMB_DIM * BW, LANES), jnp.float32),
        pltpu.VMEM((EMB_DIM, BW), jnp.float32),
        pltpu.SemaphoreType.DMA,
    ],
)
def _sc_gather(idx_hbm, tab_hbm, out_hbm, idx_v, idxrow_v, cidx_v, staged_v,
               ebuf_v, sem):
    _sc_gather_body(idx_hbm, tab_hbm, out_hbm, idx_v, idxrow_v, cidx_v,
                    staged_v, ebuf_v, sem)


def _mm_body(g_ref, ws_ref, bs_ref, d_ref, wd_ref, bd_ref, so_ref, do_ref):
    so_ref[...] = lax.dot_general(
        g_ref[...], ws_ref[...], (((0,), (1,)), ((), ())),
        preferred_element_type=jnp.float32) + bs_ref[...]
    do_ref[...] = lax.dot_general(
        d_ref[...], wd_ref[...], (((1,), (1,)), ((), ())),
        preferred_element_type=jnp.float32) + bd_ref[...]


def _tc_matmuls(g, w_s, b_s, dense, w_d, b_d):
    blk = 512
    grid = (BATCH // blk,)
    d_in = dense.shape[1]
    return pl.pallas_call(
        _mm_body,
        grid=grid,
        in_specs=[
            pl.BlockSpec((ROWS, blk), lambda i: (0, i)),
            pl.BlockSpec((w_s.shape[0], ROWS), lambda i: (0, 0)),
            pl.BlockSpec((1, w_s.shape[0]), lambda i: (0, 0)),
            pl.BlockSpec((blk, d_in), lambda i: (i, 0)),
            pl.BlockSpec((w_d.shape[0], d_in), lambda i: (0, 0)),
            pl.BlockSpec((1, w_d.shape[0]), lambda i: (0, 0)),
        ],
        out_specs=[
            pl.BlockSpec((blk, w_s.shape[0]), lambda i: (i, 0)),
            pl.BlockSpec((blk, w_d.shape[0]), lambda i: (i, 0)),
        ],
        out_shape=[
            jax.ShapeDtypeStruct((BATCH, w_s.shape[0]), jnp.float32),
            jax.ShapeDtypeStruct((BATCH, w_d.shape[0]), jnp.float32),
        ],
    )(g, w_s, b_s, dense, w_d, b_d)


def kernel(sparse_inputs, dense_inputs, tables, W_sparse, b_sparse, W_dense, b_dense):
    idx_t = jnp.transpose(sparse_inputs.astype(jnp.int32), (1, 0))
    tab = jnp.transpose(tables, (0, 2, 1)).reshape(N_CHUNKS, LANES)
    g = _sc_gather(idx_t, tab)
    sparse_out, dense_out = _tc_matmuls(
        g, W_sparse, b_sparse.reshape(1, -1),
        dense_inputs, W_dense, b_dense.reshape(1, -1))
    return (dense_out, sparse_out)
